# baseline (speedup 1.0000x reference)
import functools

import jax
import jax.numpy as jnp
from jax import lax
from jax.experimental import pallas as pl
from jax.experimental.pallas import tpu as pltpu
from jax.experimental.pallas import tpu_sc as plsc

DIM = 128
BATCH = 4096
HIST = 50
HPAD = 56
NCORES = 2
NSUB = 16
NW = NCORES * NSUB
BPW = BATCH // NW
R = 2
NCH = BPW // R
LANES = 16

_SELU_ALPHA = 1.6732632423543772
_SELU_SCALE = 1.0507009873554805


def _sc_pool(idx_flat, table):
    mesh = plsc.VectorSubcoreMesh(
        core_axis_name="c", subcore_axis_name="s",
        num_cores=NCORES, num_subcores=NSUB,
    )

    @functools.partial(
        pl.kernel,
        out_type=jax.ShapeDtypeStruct((BATCH, DIM), jnp.float32),
        mesh=mesh,
        scratch_types=[
            pltpu.VMEM((BPW * HPAD,), jnp.int32),
            pltpu.VMEM((R * HPAD, DIM), jnp.float32),
            pltpu.VMEM((R * HPAD, DIM), jnp.float32),
            pltpu.VMEM((BPW, DIM), jnp.float32),
            pltpu.SemaphoreType.DMA,
            pltpu.SemaphoreType.DMA,
        ],
    )
    def pool(idx_hbm, table_hbm, out_hbm, idx_v, buf_a, buf_b, outbuf, sem_a, sem_b):
        wid = lax.axis_index("c") * NSUB + lax.axis_index("s")
        base = wid * BPW
        pltpu.sync_copy(idx_hbm.at[pl.ds(base * HPAD, BPW * HPAD)], idx_v)

        def dma(c, buf, sem):
            return pltpu.make_async_copy(
                table_hbm.at[idx_v.at[pl.ds(c * (R * HPAD), R * HPAD)]], buf, sem
            )

        def reduce_chunk(c, buf):
            for rr in range(R):
                o = rr * HPAD

                def inner(h, accs):
                    return tuple(
                        a + buf[o + h, pl.ds(d * LANES, LANES)]
                        for d, a in enumerate(accs)
                    )

                accs = lax.fori_loop(
                    1, HIST, inner,
                    tuple(buf[o, pl.ds(d * LANES, LANES)] for d in range(8)),
                    unroll=7,
                )
                for d in range(8):
                    outbuf[c * R + rr, pl.ds(d * LANES, LANES)] = accs[d]

        dma(0, buf_a, sem_a).start()
        dma(1, buf_b, sem_b).start()

        def step(i, carry):
            c0 = 2 * i
            c1 = 2 * i + 1
            dma(c0, buf_a, sem_a).wait()
            reduce_chunk(c0, buf_a)

            @pl.when(c0 + 2 < NCH)
            def _():
                dma(c0 + 2, buf_a, sem_a).start()

            dma(c1, buf_b, sem_b).wait()
            reduce_chunk(c1, buf_b)

            @pl.when(c1 + 2 < NCH)
            def _():
                dma(c1 + 2, buf_b, sem_b).start()

            return carry

        lax.fori_loop(0, NCH // 2, step, 0)
        pltpu.sync_copy(outbuf, out_hbm.at[pl.ds(base, BPW)])

    return pool(idx_flat, table)


def _selu_linear(x, wT, b2):
    def body(x_ref, w_ref, b_ref, o_ref):
        v = x_ref[...]
        v = _SELU_SCALE * jnp.where(v > 0, v, _SELU_ALPHA * (jnp.exp(v) - 1.0))
        o_ref[...] = (
            jnp.dot(v, w_ref[...], preferred_element_type=jnp.float32) + b_ref[...]
        )

    blk = 512
    return pl.pallas_call(
        body,
        out_shape=jax.ShapeDtypeStruct((BATCH, DIM), jnp.float32),
        grid=(BATCH // blk,),
        in_specs=[
            pl.BlockSpec((blk, DIM), lambda i: (i, 0)),
            pl.BlockSpec((DIM, DIM), lambda i: (0, 0)),
            pl.BlockSpec((1, DIM), lambda i: (0, 0)),
        ],
        out_specs=pl.BlockSpec((blk, DIM), lambda i: (i, 0)),
    )(x, wT, b2)


def kernel(input_text, table, W, b):
    idx = input_text.reshape(BATCH, -1).astype(jnp.int32)
    idx = jnp.pad(idx, ((0, 0), (0, HPAD - HIST)))
    pooled = _sc_pool(idx.reshape(-1), table)
    return _selu_linear(pooled, W.T, b.reshape(1, DIM))


# speedup vs baseline: 15.3293x; 15.3293x over previous
import functools

import jax
import jax.numpy as jnp
from jax import lax
from jax.experimental import pallas as pl
from jax.experimental.pallas import tpu as pltpu
from jax.experimental.pallas import tpu_sc as plsc

DIM = 128
BATCH = 4096
HIST = 50
HPAD = 56
NCORES = 2
NSUB = 16
NW = NCORES * NSUB
BPW = BATCH // NW
NBUF = 8
LANES = 16

_SELU_ALPHA = 1.6732632423543772
_SELU_SCALE = 1.0507009873554805


def _sc_pool(idx_flat, table):
    mesh = plsc.VectorSubcoreMesh(
        core_axis_name="c", subcore_axis_name="s",
        num_cores=NCORES, num_subcores=NSUB,
    )

    @functools.partial(
        pl.kernel,
        out_type=jax.ShapeDtypeStruct((BATCH, DIM), jnp.float32),
        mesh=mesh,
        scratch_types=[
            pltpu.VMEM((BPW * HPAD,), jnp.int32),
            pltpu.VMEM((BPW, DIM), jnp.float32),
        ]
        + [pltpu.VMEM((HIST, DIM), jnp.float32)] * NBUF
        + [pltpu.SemaphoreType.DMA] * NBUF,
    )
    def pool(idx_hbm, table_hbm, out_hbm, idx_v, outbuf, *ring):
        bufs = ring[:NBUF]
        sems = ring[NBUF:]
        wid = lax.axis_index("c") * NSUB + lax.axis_index("s")
        base = wid * BPW
        pltpu.sync_copy(idx_hbm.at[pl.ds(base * HPAD, BPW * HPAD)], idx_v)

        def dma(c, b):
            return pltpu.make_async_copy(
                table_hbm.at[idx_v.at[pl.ds(c * HPAD, HIST)]], bufs[b], sems[b]
            )

        def reduce_chunk(c, b):
            buf = bufs[b]

            def inner(h, accs):
                return tuple(
                    a + buf[h, pl.ds(d * LANES, LANES)]
                    for d, a in enumerate(accs)
                )

            accs = lax.fori_loop(
                1, HIST, inner,
                tuple(buf[0, pl.ds(d * LANES, LANES)] for d in range(8)),
                unroll=7,
            )
            for d in range(8):
                outbuf[c, pl.ds(d * LANES, LANES)] = accs[d]

        for b in range(NBUF):
            dma(b, b).start()

        def step(g, carry):
            c0 = g * NBUF
            for b in range(NBUF):
                dma(c0 + b, b).wait()
                reduce_chunk(c0 + b, b)

                @pl.when(c0 + b + NBUF < BPW)
                def _():
                    dma(c0 + b + NBUF, b).start()

            return carry

        lax.fori_loop(0, BPW // NBUF, step, 0)
        pltpu.sync_copy(outbuf, out_hbm.at[pl.ds(base, BPW)])

    return pool(idx_flat, table)


def _selu_linear(x, wT, b2):
    def body(x_ref, w_ref, b_ref, o_ref):
        v = x_ref[...]
        v = _SELU_SCALE * jnp.where(v > 0, v, _SELU_ALPHA * (jnp.exp(v) - 1.0))
        o_ref[...] = (
            jnp.dot(v, w_ref[...], preferred_element_type=jnp.float32) + b_ref[...]
        )

    blk = 512
    return pl.pallas_call(
        body,
        out_shape=jax.ShapeDtypeStruct((BATCH, DIM), jnp.float32),
        grid=(BATCH // blk,),
        in_specs=[
            pl.BlockSpec((blk, DIM), lambda i: (i, 0)),
            pl.BlockSpec((DIM, DIM), lambda i: (0, 0)),
            pl.BlockSpec((1, DIM), lambda i: (0, 0)),
        ],
        out_specs=pl.BlockSpec((blk, DIM), lambda i: (i, 0)),
    )(x, wT, b2)


def kernel(input_text, table, W, b):
    idx = input_text.reshape(BATCH, -1).astype(jnp.int32)
    idx = jnp.pad(idx, ((0, 0), (0, HPAD - HIST)))
    pooled = _sc_pool(idx.reshape(-1), table)
    return _selu_linear(pooled, W.T, b.reshape(1, DIM))
